# baseline (speedup 1.0000x reference)
import functools

import jax
import jax.numpy as jnp
from jax import lax
from jax.experimental import pallas as pl
from jax.experimental.pallas import tpu as pltpu
from jax.experimental.pallas import tpu_sc as plsc

B = 1024
CAST = 1000
IN = 64
H = 64

_NC = 2
_NS = 16
_L = 16
_NW = _NC * _NS
_BPW = B // _NW


@functools.lru_cache(maxsize=None)
def _make_sc_gather():
    @functools.partial(
        pl.kernel,
        mesh=plsc.VectorSubcoreMesh(core_axis_name="c", subcore_axis_name="s"),
        out_type=jax.ShapeDtypeStruct((B, H), jnp.float32),
        scratch_types=[
            pltpu.VMEM((_BPW,), jnp.int32),
            pltpu.VMEM((_BPW, H), jnp.float32),
            pltpu.SemaphoreType.DMA,
        ],
        compiler_params=pltpu.CompilerParams(use_tc_tiling_on_sc=False),
    )
    def _sc_gather(ids_hbm, table_hbm, out_hbm, idx_v, rows_v, sem):
        wid = lax.axis_index("s") * _NC + lax.axis_index("c")
        base = wid * _BPW
        pltpu.sync_copy(ids_hbm.at[pl.ds(base, _BPW)], idx_v)
        for j in range(_BPW // _L):
            a = jnp.clip(idx_v[pl.ds(j * _L, _L)], 0, CAST - 1)
            row0 = base + j * _L
            idx_v[pl.ds(j * _L, _L)] = a + (row0 + lax.iota(jnp.int32, _L)) * CAST
        pltpu.async_copy(table_hbm.at[idx_v], rows_v, sem).wait()
        pltpu.sync_copy(rows_v, out_hbm.at[pl.ds(base, _BPW)])

    return _sc_gather


def _gru_body(x_ref, h_ref, wir, wiz, win, whr, whz, whn, brz, bin_, bhn, o_ref):
    x = x_ref[...]
    h = h_ref[...]

    def dot(a, w):
        return lax.dot_general(a, w[...], (((1,), (0,)), ((), ())),
                               preferred_element_type=jnp.float32)

    r = jax.nn.sigmoid(dot(x, wir) + dot(h, whr) + brz[0:1, :])
    z = jax.nn.sigmoid(dot(x, wiz) + dot(h, whz) + brz[1:2, :])
    hn = dot(h, whn) + bhn[...]
    n = jnp.tanh(dot(x, win) + bin_[...] + r * hn)
    o_ref[...] = (1.0 - z) * n + z * h


_gru = pl.pallas_call(
    _gru_body,
    out_shape=jax.ShapeDtypeStruct((B, H), jnp.float32),
)


def kernel(x, batch_idxs, actor_ids, story_stop_idxs, state, W_ih, W_hh, b_ih, b_hh):
    table = state.reshape(B * CAST, H)
    selected = _make_sc_gather()(actor_ids, table)
    wir, wiz, win = W_ih[:H].T, W_ih[H:2 * H].T, W_ih[2 * H:].T
    whr, whz, whn = W_hh[:H].T, W_hh[H:2 * H].T, W_hh[2 * H:].T
    brz = jnp.stack([b_ih[:H] + b_hh[:H], b_ih[H:2 * H] + b_hh[H:2 * H]])
    bin_ = b_ih[2 * H:].reshape(1, H)
    bhn = b_hh[2 * H:].reshape(1, H)
    return _gru(x, selected, wir, wiz, win, whr, whz, whn, brz, bin_, bhn)


# speedup vs baseline: 1.6127x; 1.6127x over previous
import functools

import jax
import jax.numpy as jnp
from jax import lax
from jax.experimental import pallas as pl
from jax.experimental.pallas import tpu as pltpu
from jax.experimental.pallas import tpu_sc as plsc

B = 1024
CAST = 1000
IN = 64
H = 64

_NC = 2
_NS = 16
_L = 16
_NW = _NC * _NS
_BPW = B // _NW


@functools.lru_cache(maxsize=None)
def _make_sc_gather():
    @functools.partial(
        pl.kernel,
        mesh=plsc.VectorSubcoreMesh(core_axis_name="c", subcore_axis_name="s"),
        out_type=jax.ShapeDtypeStruct((B, H), jnp.float32),
        scratch_types=[
            pltpu.VMEM((_BPW,), jnp.int32),
            pltpu.VMEM((_BPW, H), jnp.float32),
            pltpu.SemaphoreType.DMA,
        ],
        compiler_params=pltpu.CompilerParams(needs_layout_passes=False),
    )
    def _sc_gather(ids_hbm, state_hbm, out_hbm, ids_v, rows_v, sem):
        wid = lax.axis_index("s") * _NC + lax.axis_index("c")
        base = wid * _BPW
        pltpu.sync_copy(ids_hbm.at[pl.ds(base, _BPW)], ids_v)
        lanes = lax.iota(jnp.int32, _L)
        for j in range(_BPW):
            chunk = ids_v[pl.ds((j // _L) * _L, _L)]
            a = lax.reduce_max(jnp.where(lanes == (j % _L), chunk, -1), (0,))
            a = jnp.clip(a, 0, CAST - 1)
            pltpu.async_copy(state_hbm.at[base + j, a], rows_v.at[j], sem)
        for j in range(_BPW):
            pltpu.make_async_copy(state_hbm.at[0, 0], rows_v.at[0], sem).wait()
        pltpu.sync_copy(rows_v, out_hbm.at[pl.ds(base, _BPW)])

    return _sc_gather


def _gru_body(x_ref, h_ref, wir, wiz, win, whr, whz, whn, brz, bin_, bhn, o_ref):
    x = x_ref[...]
    h = h_ref[...]

    def dot(a, w):
        return lax.dot_general(a, w[...], (((1,), (0,)), ((), ())),
                               preferred_element_type=jnp.float32)

    r = jax.nn.sigmoid(dot(x, wir) + dot(h, whr) + brz[0:1, :])
    z = jax.nn.sigmoid(dot(x, wiz) + dot(h, whz) + brz[1:2, :])
    hn = dot(h, whn) + bhn[...]
    n = jnp.tanh(dot(x, win) + bin_[...] + r * hn)
    o_ref[...] = (1.0 - z) * n + z * h


_gru = pl.pallas_call(
    _gru_body,
    out_shape=jax.ShapeDtypeStruct((B, H), jnp.float32),
)


def kernel(x, batch_idxs, actor_ids, story_stop_idxs, state, W_ih, W_hh, b_ih, b_hh):
    selected = _make_sc_gather()(actor_ids, state)
    wir, wiz, win = W_ih[:H].T, W_ih[H:2 * H].T, W_ih[2 * H:].T
    whr, whz, whn = W_hh[:H].T, W_hh[H:2 * H].T, W_hh[2 * H:].T
    brz = jnp.stack([b_ih[:H] + b_hh[:H], b_ih[H:2 * H] + b_hh[H:2 * H]])
    bin_ = b_ih[2 * H:].reshape(1, H)
    bhn = b_hh[2 * H:].reshape(1, H)
    return _gru(x, selected, wir, wiz, win, whr, whz, whn, brz, bin_, bhn)


# speedup vs baseline: 18.6005x; 11.5337x over previous
import functools

import jax
import jax.numpy as jnp
from jax import lax
from jax.experimental import pallas as pl
from jax.experimental.pallas import tpu as pltpu
from jax.experimental.pallas import tpu_sc as plsc

B = 1024
CAST = 1000
IN = 64
H = 64

_NC = 2
_NS = 16
_L = 16
_NW = _NC * _NS
_BPW = B // _NW


@functools.lru_cache(maxsize=None)
def _make_sc_gather():
    @functools.partial(
        pl.kernel,
        mesh=plsc.VectorSubcoreMesh(core_axis_name="c", subcore_axis_name="s"),
        out_type=jax.ShapeDtypeStruct((B * H,), jnp.float32),
        scratch_types=[
            pltpu.VMEM((_BPW,), jnp.int32),
            pltpu.VMEM((_BPW * H // 128, 128), jnp.int32),
            pltpu.VMEM((_BPW * H // 128, 128), jnp.float32),
            pltpu.SemaphoreType.DMA,
        ],
        compiler_params=pltpu.CompilerParams(needs_layout_passes=False),
    )
    def _sc_gather(ids_hbm, flat_hbm, out_hbm, ids_v, idx_v, rows_v, sem):
        wid = lax.axis_index("s") * _NC + lax.axis_index("c")
        base = wid * _BPW
        pltpu.sync_copy(ids_hbm.at[pl.ds(base, _BPW)], ids_v)
        lanes = lax.iota(jnp.int32, _L)
        for h in range(H):
            for half in range(_BPW // _L):
                a = jnp.clip(ids_v[pl.ds(half * _L, _L)], 0, CAST - 1)
                i = base + half * _L + lanes
                idx = a * (H * 1024) + (h // 8) * 8192 + (i // 128) * 1024 \
                    + (h % 8) * 128 + (i % 128)
                g = h * (_BPW // _L) + half
                idx_v[g // 8, pl.ds((g % 8) * _L, _L)] = idx
        n_chunks = _BPW * H // 128
        descs = [pltpu.async_copy(flat_hbm.at[idx_v.at[q]], rows_v.at[q], sem)
                 for q in range(n_chunks)]
        for d in descs:
            d.wait()
        for q in range(n_chunks):
            pltpu.sync_copy(rows_v.at[q], out_hbm.at[pl.ds(base * H + q * 128, 128)])

    return _sc_gather


def _gru_body(x_ref, h_ref, wir, wiz, win, whr, whz, whn, brz, bin_, bhn, o_ref):
    x = x_ref[...]
    h = h_ref[...]

    def dot(a, w):
        return lax.dot_general(a, w[...], (((1,), (0,)), ((), ())),
                               preferred_element_type=jnp.float32)

    r = jax.nn.sigmoid(dot(x, wir) + dot(h, whr) + brz[0:1, :])
    z = jax.nn.sigmoid(dot(x, wiz) + dot(h, whz) + brz[1:2, :])
    hn = dot(h, whn) + bhn[...]
    n = jnp.tanh(dot(x, win) + bin_[...] + r * hn)
    o_ref[...] = (1.0 - z) * n + z * h


_gru = pl.pallas_call(
    _gru_body,
    out_shape=jax.ShapeDtypeStruct((B, H), jnp.float32),
)


def kernel(x, batch_idxs, actor_ids, story_stop_idxs, state, W_ih, W_hh, b_ih, b_hh):
    st_t = jnp.transpose(state, (1, 2, 0))
    st_5d = st_t.reshape(CAST, 8, H // 8, 8, 128)
    st_phys = jnp.transpose(st_5d, (0, 1, 3, 2, 4))
    flat = st_phys.reshape(-1)
    gathered = _make_sc_gather()(actor_ids, flat)
    selected = gathered.reshape(_NW, H, _BPW).transpose(0, 2, 1).reshape(B, H)
    wir, wiz, win = W_ih[:H].T, W_ih[H:2 * H].T, W_ih[2 * H:].T
    whr, whz, whn = W_hh[:H].T, W_hh[H:2 * H].T, W_hh[2 * H:].T
    brz = jnp.stack([b_ih[:H] + b_hh[:H], b_ih[H:2 * H] + b_hh[H:2 * H]])
    bin_ = b_ih[2 * H:].reshape(1, H)
    bhn = b_hh[2 * H:].reshape(1, H)
    return _gru(x, selected, wir, wiz, win, whr, whz, whn, brz, bin_, bhn)


# speedup vs baseline: 21.4558x; 1.1535x over previous
import functools

import jax
import jax.numpy as jnp
from jax import lax
from jax.experimental import pallas as pl
from jax.experimental.pallas import tpu as pltpu
from jax.experimental.pallas import tpu_sc as plsc

B = 1024
CAST = 1000
IN = 64
H = 64

_NC = 2
_NS = 16
_L = 16
_NW = _NC * _NS
_BPW = B // _NW
_NCHUNK = _BPW * H // 128


@functools.lru_cache(maxsize=None)
def _make_sc_gather():
    @functools.partial(
        pl.kernel,
        mesh=plsc.VectorSubcoreMesh(core_axis_name="c", subcore_axis_name="s"),
        out_type=jax.ShapeDtypeStruct((_NW, _NCHUNK, 128), jnp.float32),
        scratch_types=[
            pltpu.VMEM((_BPW,), jnp.int32),
            pltpu.VMEM((_NCHUNK, 128), jnp.int32),
            pltpu.VMEM((_NCHUNK, 128), jnp.float32),
            pltpu.SemaphoreType.DMA,
        ],
        compiler_params=pltpu.CompilerParams(needs_layout_passes=False),
    )
    def _sc_gather(ids_hbm, flat_hbm, out_hbm, ids_v, idx_v, rows_v, sem):
        wid = lax.axis_index("s") * _NC + lax.axis_index("c")
        base = wid * _BPW
        pltpu.sync_copy(ids_hbm.at[pl.ds(base, _BPW)], ids_v)
        lanes = lax.iota(jnp.int32, _L)
        av = []
        for half in range(_BPW // _L):
            a = jnp.clip(ids_v[pl.ds(half * _L, _L)], 0, CAST - 1)
            i = base + half * _L + lanes
            av.append(a * (H * 1024) + (i // 128) * 1024 + (i % 128))
        for h in range(H):
            hoff = (h // 8) * 8192 + (h % 8) * 128
            for half in range(_BPW // _L):
                g = h * (_BPW // _L) + half
                idx_v[g // 8, pl.ds((g % 8) * _L, _L)] = av[half] + hoff
        descs = [pltpu.async_copy(flat_hbm.at[idx_v.at[q]], rows_v.at[q], sem)
                 for q in range(_NCHUNK)]
        for d in descs:
            d.wait()
        pltpu.sync_copy(rows_v, out_hbm.at[wid])

    return _sc_gather


def _gru_body(xt_ref, ht_ref, wir, wiz, win, whr, whz, whn, br, bz, bin_, bhn,
              ot_ref):
    xt = xt_ref[...]
    ht = ht_ref[...]

    def dot(w, v):
        return lax.dot_general(w[...], v, (((1,), (0,)), ((), ())),
                               preferred_element_type=jnp.float32)

    r = jax.nn.sigmoid(dot(wir, xt) + dot(whr, ht) + br[...])
    z = jax.nn.sigmoid(dot(wiz, xt) + dot(whz, ht) + bz[...])
    hn = dot(whn, ht) + bhn[...]
    n = jnp.tanh(dot(win, xt) + bin_[...] + r * hn)
    ot_ref[...] = (1.0 - z) * n + z * ht


_gru = pl.pallas_call(
    _gru_body,
    out_shape=jax.ShapeDtypeStruct((H, B), jnp.float32),
)


def kernel(x, batch_idxs, actor_ids, story_stop_idxs, state, W_ih, W_hh, b_ih, b_hh):
    st_t = jnp.transpose(state, (1, 2, 0))
    st_5d = st_t.reshape(CAST, 8, H // 8, 8, 128)
    st_phys = jnp.transpose(st_5d, (0, 1, 3, 2, 4))
    flat = st_phys.reshape(-1)
    g = _make_sc_gather()(actor_ids, flat)
    ht = g.reshape(_NW, H, _BPW).transpose(1, 0, 2).reshape(H, B)
    xt = x.T
    wir, wiz, win = W_ih[:H], W_ih[H:2 * H], W_ih[2 * H:]
    whr, whz, whn = W_hh[:H], W_hh[H:2 * H], W_hh[2 * H:]
    br = (b_ih[:H] + b_hh[:H]).reshape(H, 1)
    bz = (b_ih[H:2 * H] + b_hh[H:2 * H]).reshape(H, 1)
    bin_ = b_ih[2 * H:].reshape(H, 1)
    bhn = b_hh[2 * H:].reshape(H, 1)
    out_t = _gru(xt, ht, wir, wiz, win, whr, whz, whn, br, bz, bin_, bhn)
    return out_t.T


# speedup vs baseline: 21.5279x; 1.0034x over previous
import functools

import jax
import jax.numpy as jnp
from jax import lax
from jax.experimental import pallas as pl
from jax.experimental.pallas import tpu as pltpu
from jax.experimental.pallas import tpu_sc as plsc

B = 1024
CAST = 1000
IN = 64
H = 64

_NC = 2
_NS = 16
_L = 16
_NW = _NC * _NS
_BPW = B // _NW
_NCHUNK = _BPW * H // 128


@functools.lru_cache(maxsize=None)
def _make_sc_gather():
    @functools.partial(
        pl.kernel,
        mesh=plsc.VectorSubcoreMesh(core_axis_name="c", subcore_axis_name="s"),
        out_type=jax.ShapeDtypeStruct((_NW, _NCHUNK, 128), jnp.float32),
        scratch_types=[
            pltpu.VMEM((_BPW,), jnp.int32),
            pltpu.VMEM((_NCHUNK, 128), jnp.int32),
            pltpu.VMEM((_NCHUNK, 128), jnp.float32),
            pltpu.SemaphoreType.DMA,
        ],
        compiler_params=pltpu.CompilerParams(needs_layout_passes=False),
    )
    def _sc_gather(ids_hbm, flat_hbm, out_hbm, ids_v, idx_v, rows_v, sem):
        wid = lax.axis_index("s") * _NC + lax.axis_index("c")
        base = wid * _BPW
        pltpu.sync_copy(ids_hbm.at[pl.ds(base, _BPW)], ids_v)
        lanes = lax.iota(jnp.int32, _L)
        av = []
        for half in range(_BPW // _L):
            a = jnp.clip(ids_v[pl.ds(half * _L, _L)], 0, CAST - 1)
            i = base + half * _L + lanes
            av.append(a * (H * 1024) + (i // 128) * 1024 + (i % 128))
        for h in range(H):
            hoff = (h // 8) * 8192 + (h % 8) * 128
            for half in range(_BPW // _L):
                g = h * (_BPW // _L) + half
                idx_v[g // 8, pl.ds((g % 8) * _L, _L)] = av[half] + hoff
        descs = [pltpu.async_copy(flat_hbm.at[idx_v.at[q]], rows_v.at[q], sem)
                 for q in range(_NCHUNK)]
        for d in descs:
            d.wait()
        pltpu.sync_copy(rows_v, out_hbm.at[wid])

    return _sc_gather


def _gru_body(xt_ref, ht_ref, wih_t, whh_t, br, bz, bin_, bhn, ot_ref):
    xt = xt_ref[...]
    ht = ht_ref[...]
    wi = wih_t[...]
    wh = whh_t[...]

    def dot(w, lo, v):
        return lax.dot_general(w[:, lo:lo + H], v, (((0,), (0,)), ((), ())),
                               preferred_element_type=jnp.float32)

    r = jax.nn.sigmoid(dot(wi, 0, xt) + dot(wh, 0, ht) + br[...])
    z = jax.nn.sigmoid(dot(wi, H, xt) + dot(wh, H, ht) + bz[...])
    hn = dot(wh, 2 * H, ht) + bhn[...]
    n = jnp.tanh(dot(wi, 2 * H, xt) + bin_[...] + r * hn)
    ot_ref[...] = (1.0 - z) * n + z * ht


_gru = pl.pallas_call(
    _gru_body,
    out_shape=jax.ShapeDtypeStruct((H, B), jnp.float32),
)


def kernel(x, batch_idxs, actor_ids, story_stop_idxs, state, W_ih, W_hh, b_ih, b_hh):
    st_t = jnp.transpose(state, (1, 2, 0))
    st_5d = st_t.reshape(CAST, 8, H // 8, 8, 128)
    st_phys = jnp.transpose(st_5d, (0, 1, 3, 2, 4))
    flat = st_phys.reshape(-1)
    g = _make_sc_gather()(actor_ids, flat)
    ht = g.reshape(_NW, H, _BPW).transpose(1, 0, 2).reshape(H, B)
    xt = x.T
    br = (b_ih[:H] + b_hh[:H]).reshape(H, 1)
    bz = (b_ih[H:2 * H] + b_hh[H:2 * H]).reshape(H, 1)
    bin_ = b_ih[2 * H:].reshape(H, 1)
    bhn = b_hh[2 * H:].reshape(H, 1)
    out_t = _gru(xt, ht, W_ih.T, W_hh.T, br, bz, bin_, bhn)
    return out_t.T


# speedup vs baseline: 21.8001x; 1.0126x over previous
import functools

import jax
import jax.numpy as jnp
from jax import lax
from jax.experimental import pallas as pl
from jax.experimental.pallas import tpu as pltpu
from jax.experimental.pallas import tpu_sc as plsc

B = 1024
CAST = 1000
IN = 64
H = 64

_NC = 2
_NS = 16
_L = 16
_NW = _NC * _NS
_BPW = B // _NW
_NCHUNK = _BPW * H // 128


@functools.lru_cache(maxsize=None)
def _make_sc_gather():
    @functools.partial(
        pl.kernel,
        mesh=plsc.VectorSubcoreMesh(core_axis_name="c", subcore_axis_name="s"),
        out_type=jax.ShapeDtypeStruct((_NW, _NCHUNK, 128), jnp.float32),
        scratch_types=[
            pltpu.VMEM((_BPW,), jnp.int32),
            pltpu.VMEM((_NCHUNK, 128), jnp.int32),
            pltpu.VMEM((_NCHUNK, 128), jnp.float32),
            pltpu.SemaphoreType.DMA,
        ],
        compiler_params=pltpu.CompilerParams(needs_layout_passes=False,
                                             skip_device_barrier=True),
    )
    def _sc_gather(ids_hbm, flat_hbm, out_hbm, ids_v, idx_v, rows_v, sem):
        wid = lax.axis_index("s") * _NC + lax.axis_index("c")
        base = wid * _BPW
        pltpu.sync_copy(ids_hbm.at[pl.ds(base, _BPW)], ids_v)
        lanes = lax.iota(jnp.int32, _L)
        av = []
        for half in range(_BPW // _L):
            a = jnp.clip(ids_v[pl.ds(half * _L, _L)], 0, CAST - 1)
            i = base + half * _L + lanes
            av.append(a * (H * 1024) + (i // 128) * 1024 + (i % 128))
        for h in range(H):
            hoff = (h // 8) * 8192 + (h % 8) * 128
            for half in range(_BPW // _L):
                g = h * (_BPW // _L) + half
                idx_v[g // 8, pl.ds((g % 8) * _L, _L)] = av[half] + hoff
        descs = [pltpu.async_copy(flat_hbm.at[idx_v.at[q]], rows_v.at[q], sem)
                 for q in range(_NCHUNK)]
        for d in descs:
            d.wait()
        pltpu.sync_copy(rows_v, out_hbm.at[wid])

    return _sc_gather


def _gru_body(xt_ref, ht_ref, wih_t, whh_t, br, bz, bin_, bhn, ot_ref):
    xt = xt_ref[...]
    ht = ht_ref[...]
    wi = wih_t[...]
    wh = whh_t[...]

    def dot(w, lo, v):
        return lax.dot_general(w[:, lo:lo + H], v, (((0,), (0,)), ((), ())),
                               preferred_element_type=jnp.float32)

    r = jax.nn.sigmoid(dot(wi, 0, xt) + dot(wh, 0, ht) + br[...])
    z = jax.nn.sigmoid(dot(wi, H, xt) + dot(wh, H, ht) + bz[...])
    hn = dot(wh, 2 * H, ht) + bhn[...]
    n = jnp.tanh(dot(wi, 2 * H, xt) + bin_[...] + r * hn)
    ot_ref[...] = (1.0 - z) * n + z * ht


_gru = pl.pallas_call(
    _gru_body,
    out_shape=jax.ShapeDtypeStruct((H, B), jnp.float32),
)


def kernel(x, batch_idxs, actor_ids, story_stop_idxs, state, W_ih, W_hh, b_ih, b_hh):
    st_t = jnp.transpose(state, (1, 2, 0))
    st_5d = st_t.reshape(CAST, 8, H // 8, 8, 128)
    st_phys = jnp.transpose(st_5d, (0, 1, 3, 2, 4))
    flat = st_phys.reshape(-1)
    g = _make_sc_gather()(actor_ids, flat)
    ht = g.reshape(_NW, H, _BPW).transpose(1, 0, 2).reshape(H, B)
    xt = x.T
    br = (b_ih[:H] + b_hh[:H]).reshape(H, 1)
    bz = (b_ih[H:2 * H] + b_hh[H:2 * H]).reshape(H, 1)
    bin_ = b_ih[2 * H:].reshape(H, 1)
    bhn = b_hh[2 * H:].reshape(H, 1)
    out_t = _gru(xt, ht, W_ih.T, W_hh.T, br, bz, bin_, bhn)
    return out_t.T
